# baseline (speedup 1.0000x reference)
import jax
import jax.numpy as jnp
import numpy as np
from jax import lax
from jax.experimental import pallas as pl
from jax.experimental.pallas import tpu as pltpu

_ROWS, _VOCAB = 64, 100000
_CHUNK = 25600
_GRID = (_VOCAB + _CHUNK - 1) // _CHUNK
_NEG_INF = float(np.finfo(np.float32).min)
_BIG_I32 = np.int32(2**31 - 1)


def _rotl(x, r):
    return ((x << np.uint32(r)) | (x >> np.uint32(32 - r))).astype(np.uint32)


def _threefry2x32(k0, k1, x0, x1):
    ks = [np.uint32(k0), np.uint32(k1),
          np.uint32(k0) ^ np.uint32(k1) ^ np.uint32(0x1BD11BDA)]
    x0 = (x0 + ks[0]).astype(np.uint32)
    x1 = (x1 + ks[1]).astype(np.uint32)
    rot = [[13, 15, 26, 6], [17, 29, 16, 24]]
    for i in range(5):
        for r in rot[i % 2]:
            x0 = (x0 + x1).astype(np.uint32)
            x1 = _rotl(x1, r)
            x1 = (x1 ^ x0).astype(np.uint32)
        x0 = (x0 + ks[(i + 1) % 3]).astype(np.uint32)
        x1 = (x1 + ks[(i + 2) % 3] + np.uint32(i + 1)).astype(np.uint32)
    return x0, x1


def _race_offsets():
    i64 = np.arange(_ROWS * _VOCAB, dtype=np.uint64)
    b1, b2 = _threefry2x32(0, 1234,
                           (i64 >> np.uint64(32)).astype(np.uint32),
                           (i64 & np.uint64(0xFFFFFFFF)).astype(np.uint32))
    bits = (b1 ^ b2).astype(np.uint32)
    fb = (bits >> np.uint32(9)) | np.uint32(0x3F800000)
    u = np.maximum(np.float32(0.0), fb.view(np.float32) - np.float32(1.0))
    noise = np.maximum(-np.log1p(-u), np.float32(1e-10))
    return (-np.log(noise.astype(np.float64))).astype(np.float32).reshape(
        _ROWS, _VOCAB)


_G = _race_offsets()


def _quantize_g():
    lo = _G.min(axis=1)
    hi = _G.max(axis=1)
    scale = ((hi - lo).astype(np.float64) / 65534.0).astype(np.float32)
    q = np.ceil((_G - lo[:, None]).astype(np.float64)
                / scale[:, None].astype(np.float64)).astype(np.int64)
    q = np.clip(q, 0, 65534)
    for _ in range(2):
        dec = (lo[:, None] + scale[:, None]
               * q.astype(np.float32)).astype(np.float32)
        q = np.where(dec < _G, q + 1, q)
        q = np.clip(q, 0, 65535)
    return q.astype(np.uint16), lo.reshape(-1, 1), scale.reshape(-1, 1)


_G16, _GLO, _GSCALE = _quantize_g()


def _scan_body(t_ref, lo_ref, sc_ref, x_ref, q_ref,
               om_ref, oi_ref, ol_ref, os_ref,
               m1_sc, i1_sc, lv_sc, m2_sc):
    j = pl.program_id(0)
    t = t_ref[:, :]
    invt = 1.0 / jnp.where(t == 0.0, 1.0, t)
    gscale = jnp.where(jnp.all(t == 0.0), 0.0, 1.0)
    glo = lo_ref[:, :] * gscale
    gsc = sc_ref[:, :] * gscale

    xi = x_ref[:, :] * invt
    qf = q_ref[:, :].astype(jnp.float32)
    col = jax.lax.broadcasted_iota(jnp.int32, xi.shape, 1)
    u = xi + (qf * gsc + glo)
    u = jnp.where(col + j * _CHUNK < _VOCAB, u, _NEG_INF)

    bm = jnp.max(u, axis=1, keepdims=True)
    ba = jnp.min(jnp.where(u == bm, col, _BIG_I32),
                 axis=1, keepdims=True)
    at_arg = col == ba
    bl = jnp.max(jnp.where(at_arg, xi, _NEG_INF),
                 axis=1, keepdims=True)
    bs = jnp.max(jnp.where(at_arg, _NEG_INF, u),
                 axis=1, keepdims=True)
    ba = ba + j * _CHUNK

    @pl.when(j == 0)
    def _():
        m1_sc[:, :] = bm
        i1_sc[:, :] = ba
        lv_sc[:, :] = bl
        m2_sc[:, :] = bs

    @pl.when(j > 0)
    def _():
        m1 = m1_sc[:, :]
        upd = bm > m1
        m2_sc[:, :] = jnp.where(upd, jnp.maximum(m1, bs),
                                jnp.maximum(m2_sc[:, :], bm))
        m1_sc[:, :] = jnp.where(upd, bm, m1)
        i1_sc[:, :] = jnp.where(upd, ba, i1_sc[:, :])
        lv_sc[:, :] = jnp.where(upd, bl, lv_sc[:, :])

    @pl.when(j == _GRID - 1)
    def _():
        om_ref[:, :] = m1_sc[:, :]
        oi_ref[:, :] = i1_sc[:, :]
        ol_ref[:, :] = lv_sc[:, :]
        os_ref[:, :] = m2_sc[:, :]


def _race_scan(t2, logits):
    return pl.pallas_call(
        _scan_body,
        grid=(_GRID,),
        in_specs=[
            pl.BlockSpec((_ROWS, 1), lambda j: (0, 0)),
            pl.BlockSpec((_ROWS, 1), lambda j: (0, 0)),
            pl.BlockSpec((_ROWS, 1), lambda j: (0, 0)),
            pl.BlockSpec((_ROWS, _CHUNK), lambda j: (0, j)),
            pl.BlockSpec((_ROWS, _CHUNK), lambda j: (0, j)),
        ],
        out_specs=[
            pl.BlockSpec((_ROWS, 1), lambda j: (0, 0)),
            pl.BlockSpec((_ROWS, 1), lambda j: (0, 0)),
            pl.BlockSpec((_ROWS, 1), lambda j: (0, 0)),
            pl.BlockSpec((_ROWS, 1), lambda j: (0, 0)),
        ],
        out_shape=[
            jax.ShapeDtypeStruct((_ROWS, 1), jnp.float32),
            jax.ShapeDtypeStruct((_ROWS, 1), jnp.int32),
            jax.ShapeDtypeStruct((_ROWS, 1), jnp.float32),
            jax.ShapeDtypeStruct((_ROWS, 1), jnp.float32),
        ],
        scratch_shapes=[
            pltpu.VMEM((_ROWS, 1), jnp.float32),
            pltpu.VMEM((_ROWS, 1), jnp.int32),
            pltpu.VMEM((_ROWS, 1), jnp.float32),
            pltpu.VMEM((_ROWS, 1), jnp.float32),
        ],
    )(t2, jnp.asarray(_GLO), jnp.asarray(_GSCALE), logits, jnp.asarray(_G16))


def _race_body(t_ref, x_ref, g_ref, o_ref, m_sc, i_sc):
    j = pl.program_id(0)
    t = t_ref[:, :]
    invt = 1.0 / jnp.where(t == 0.0, 1.0, t)
    gscale = jnp.where(jnp.all(t == 0.0), 0.0, 1.0)
    x = x_ref[:, :]
    g = g_ref[:, :]
    col = jax.lax.broadcasted_iota(jnp.int32, x.shape, 1)
    val = x * invt + g * gscale
    val = jnp.where(col + j * _CHUNK < _VOCAB, val, _NEG_INF)
    bmax = jnp.max(val, axis=1, keepdims=True)
    barg = jnp.min(jnp.where(val == bmax, col, _BIG_I32),
                   axis=1, keepdims=True) + j * _CHUNK

    @pl.when(j == 0)
    def _():
        m_sc[:, :] = jnp.full_like(bmax, _NEG_INF)
        i_sc[:, :] = jnp.zeros_like(barg)

    upd = bmax > m_sc[:, :]
    m_sc[:, :] = jnp.where(upd, bmax, m_sc[:, :])
    i_sc[:, :] = jnp.where(upd, barg, i_sc[:, :])

    @pl.when(j == _GRID - 1)
    def _():
        o_ref[:, :] = i_sc[:, :]


def _dense_race(t2, logits):
    out = pl.pallas_call(
        _race_body,
        grid=(_GRID,),
        in_specs=[
            pl.BlockSpec((_ROWS, 1), lambda j: (0, 0)),
            pl.BlockSpec((_ROWS, _CHUNK), lambda j: (0, j)),
            pl.BlockSpec((_ROWS, _CHUNK), lambda j: (0, j)),
        ],
        out_specs=pl.BlockSpec((_ROWS, 1), lambda j: (0, 0)),
        out_shape=jax.ShapeDtypeStruct((_ROWS, 1), jnp.int32),
        scratch_shapes=[
            pltpu.VMEM((_ROWS, 1), jnp.float32),
            pltpu.VMEM((_ROWS, 1), jnp.int32),
        ],
    )(t2, logits, jnp.asarray(_G))
    return out[:, 0]


def kernel(logits, temperatures):
    t = temperatures.astype(jnp.float32)
    t2 = t.reshape(_ROWS, 1)
    u1, i1, lv, u2 = _race_scan(t2, logits)
    idx = i1[:, 0]
    gscale = jnp.where(jnp.all(t == 0.0), 0.0, 1.0)
    g_exact = jnp.take_along_axis(jnp.asarray(_G), i1, axis=1)[:, 0]
    val_exact = lv[:, 0] + g_exact * gscale
    safe = jnp.all(val_exact > u2[:, 0])
    return lax.cond(safe,
                    lambda: idx,
                    lambda: _dense_race(t2, logits))


# speedup vs baseline: 1.0282x; 1.0282x over previous
import jax
import jax.numpy as jnp
import numpy as np
from jax import lax
from jax.experimental import pallas as pl
from jax.experimental.pallas import tpu as pltpu

_ROWS, _VOCAB = 64, 100000
_CHUNK = 25600
_GRID = (_VOCAB + _CHUNK - 1) // _CHUNK
_NEG_INF = float(np.finfo(np.float32).min)
_BIG_I32 = np.int32(2**31 - 1)


def _rotl(x, r):
    return ((x << np.uint32(r)) | (x >> np.uint32(32 - r))).astype(np.uint32)


def _threefry2x32(k0, k1, x0, x1):
    ks = [np.uint32(k0), np.uint32(k1),
          np.uint32(k0) ^ np.uint32(k1) ^ np.uint32(0x1BD11BDA)]
    x0 = (x0 + ks[0]).astype(np.uint32)
    x1 = (x1 + ks[1]).astype(np.uint32)
    rot = [[13, 15, 26, 6], [17, 29, 16, 24]]
    for i in range(5):
        for r in rot[i % 2]:
            x0 = (x0 + x1).astype(np.uint32)
            x1 = _rotl(x1, r)
            x1 = (x1 ^ x0).astype(np.uint32)
        x0 = (x0 + ks[(i + 1) % 3]).astype(np.uint32)
        x1 = (x1 + ks[(i + 2) % 3] + np.uint32(i + 1)).astype(np.uint32)
    return x0, x1


def _race_offsets():
    i64 = np.arange(_ROWS * _VOCAB, dtype=np.uint64)
    b1, b2 = _threefry2x32(0, 1234,
                           (i64 >> np.uint64(32)).astype(np.uint32),
                           (i64 & np.uint64(0xFFFFFFFF)).astype(np.uint32))
    bits = (b1 ^ b2).astype(np.uint32)
    fb = (bits >> np.uint32(9)) | np.uint32(0x3F800000)
    u = np.maximum(np.float32(0.0), fb.view(np.float32) - np.float32(1.0))
    noise = np.maximum(-np.log1p(-u), np.float32(1e-10))
    return (-np.log(noise.astype(np.float64))).astype(np.float32).reshape(
        _ROWS, _VOCAB)


_G = _race_offsets()


def _quantize_g():
    lo = _G.min(axis=1)
    hi = _G.max(axis=1)
    scale = ((hi - lo).astype(np.float64) / 65534.0).astype(np.float32)
    q = np.ceil((_G - lo[:, None]).astype(np.float64)
                / scale[:, None].astype(np.float64)).astype(np.int64)
    q = np.clip(q, 0, 65534)
    for _ in range(2):
        dec = (lo[:, None] + scale[:, None]
               * q.astype(np.float32)).astype(np.float32)
        q = np.where(dec < _G, q + 1, q)
        q = np.clip(q, 0, 65535)
    return q.astype(np.uint16), lo.reshape(-1, 1), scale.reshape(-1, 1)


_G16, _GLO, _GSCALE = _quantize_g()


def _scan_body(t_ref, lo_ref, sc_ref, x_ref, q_ref,
               om_ref, oi_ref, os_ref,
               m1_sc, i1_sc, m2_sc):
    j = pl.program_id(0)
    t = t_ref[:, :]
    invt = 1.0 / jnp.where(t == 0.0, 1.0, t)
    gscale = jnp.where(jnp.all(t == 0.0), 0.0, 1.0)
    glo = lo_ref[:, :] * gscale
    gsc = sc_ref[:, :] * gscale

    xi = x_ref[:, :] * invt
    qf = q_ref[:, :].astype(jnp.float32)
    col = jax.lax.broadcasted_iota(jnp.int32, xi.shape, 1)
    u = xi + (qf * gsc + glo)
    u = jnp.where(col + j * _CHUNK < _VOCAB, u, _NEG_INF)

    bm = jnp.max(u, axis=1, keepdims=True)
    ba = jnp.min(jnp.where(u == bm, col, _BIG_I32),
                 axis=1, keepdims=True)
    bs = jnp.max(jnp.where(col == ba, _NEG_INF, u),
                 axis=1, keepdims=True)
    ba = ba + j * _CHUNK

    @pl.when(j == 0)
    def _():
        m1_sc[:, :] = bm
        i1_sc[:, :] = ba
        m2_sc[:, :] = bs

    @pl.when(j > 0)
    def _():
        m1 = m1_sc[:, :]
        upd = bm > m1
        m2_sc[:, :] = jnp.where(upd, jnp.maximum(m1, bs),
                                jnp.maximum(m2_sc[:, :], bm))
        m1_sc[:, :] = jnp.where(upd, bm, m1)
        i1_sc[:, :] = jnp.where(upd, ba, i1_sc[:, :])

    @pl.when(j == _GRID - 1)
    def _():
        om_ref[:, :] = m1_sc[:, :]
        oi_ref[:, :] = i1_sc[:, :]
        os_ref[:, :] = m2_sc[:, :]


def _race_scan(t2, logits):
    return pl.pallas_call(
        _scan_body,
        grid=(_GRID,),
        in_specs=[
            pl.BlockSpec((_ROWS, 1), lambda j: (0, 0)),
            pl.BlockSpec((_ROWS, 1), lambda j: (0, 0)),
            pl.BlockSpec((_ROWS, 1), lambda j: (0, 0)),
            pl.BlockSpec((_ROWS, _CHUNK), lambda j: (0, j)),
            pl.BlockSpec((_ROWS, _CHUNK), lambda j: (0, j)),
        ],
        out_specs=[
            pl.BlockSpec((_ROWS, 1), lambda j: (0, 0)),
            pl.BlockSpec((_ROWS, 1), lambda j: (0, 0)),
            pl.BlockSpec((_ROWS, 1), lambda j: (0, 0)),
        ],
        out_shape=[
            jax.ShapeDtypeStruct((_ROWS, 1), jnp.float32),
            jax.ShapeDtypeStruct((_ROWS, 1), jnp.int32),
            jax.ShapeDtypeStruct((_ROWS, 1), jnp.float32),
        ],
        scratch_shapes=[
            pltpu.VMEM((_ROWS, 1), jnp.float32),
            pltpu.VMEM((_ROWS, 1), jnp.int32),
            pltpu.VMEM((_ROWS, 1), jnp.float32),
        ],
    )(t2, jnp.asarray(_GLO), jnp.asarray(_GSCALE), logits, jnp.asarray(_G16))


def _race_body(t_ref, x_ref, g_ref, o_ref, m_sc, i_sc):
    j = pl.program_id(0)
    t = t_ref[:, :]
    invt = 1.0 / jnp.where(t == 0.0, 1.0, t)
    gscale = jnp.where(jnp.all(t == 0.0), 0.0, 1.0)
    x = x_ref[:, :]
    g = g_ref[:, :]
    col = jax.lax.broadcasted_iota(jnp.int32, x.shape, 1)
    val = x * invt + g * gscale
    val = jnp.where(col + j * _CHUNK < _VOCAB, val, _NEG_INF)
    bmax = jnp.max(val, axis=1, keepdims=True)
    barg = jnp.min(jnp.where(val == bmax, col, _BIG_I32),
                   axis=1, keepdims=True) + j * _CHUNK

    @pl.when(j == 0)
    def _():
        m_sc[:, :] = jnp.full_like(bmax, _NEG_INF)
        i_sc[:, :] = jnp.zeros_like(barg)

    upd = bmax > m_sc[:, :]
    m_sc[:, :] = jnp.where(upd, bmax, m_sc[:, :])
    i_sc[:, :] = jnp.where(upd, barg, i_sc[:, :])

    @pl.when(j == _GRID - 1)
    def _():
        o_ref[:, :] = i_sc[:, :]


def _dense_race(t2, logits):
    out = pl.pallas_call(
        _race_body,
        grid=(_GRID,),
        in_specs=[
            pl.BlockSpec((_ROWS, 1), lambda j: (0, 0)),
            pl.BlockSpec((_ROWS, _CHUNK), lambda j: (0, j)),
            pl.BlockSpec((_ROWS, _CHUNK), lambda j: (0, j)),
        ],
        out_specs=pl.BlockSpec((_ROWS, 1), lambda j: (0, 0)),
        out_shape=jax.ShapeDtypeStruct((_ROWS, 1), jnp.int32),
        scratch_shapes=[
            pltpu.VMEM((_ROWS, 1), jnp.float32),
            pltpu.VMEM((_ROWS, 1), jnp.int32),
        ],
    )(t2, logits, jnp.asarray(_G))
    return out[:, 0]


def kernel(logits, temperatures):
    t = temperatures.astype(jnp.float32)
    t2 = t.reshape(_ROWS, 1)
    u1, i1, u2 = _race_scan(t2, logits)
    idx = i1[:, 0]
    gscale = jnp.where(jnp.all(t == 0.0), 0.0, 1.0)
    invt = 1.0 / jnp.where(t == 0.0, 1.0, t)
    g_exact = jnp.take_along_axis(jnp.asarray(_G), i1, axis=1)[:, 0]
    l_exact = jnp.take_along_axis(logits, i1, axis=1)[:, 0] * invt
    val_exact = l_exact + g_exact * gscale
    safe = jnp.all(val_exact > u2[:, 0])
    return lax.cond(safe,
                    lambda: idx,
                    lambda: _dense_race(t2, logits))


# speedup vs baseline: 1.2246x; 1.1910x over previous
import jax
import jax.numpy as jnp
import numpy as np
from jax import lax
from jax.experimental import pallas as pl
from jax.experimental.pallas import tpu as pltpu

_ROWS, _VOCAB = 64, 100000
_CHUNK = 25600
_GRID = (_VOCAB + _CHUNK - 1) // _CHUNK
_NEG_INF = float(np.finfo(np.float32).min)
_BIG_I32 = np.int32(2**31 - 1)


def _rotl(x, r):
    return ((x << np.uint32(r)) | (x >> np.uint32(32 - r))).astype(np.uint32)


def _threefry2x32(k0, k1, x0, x1):
    ks = [np.uint32(k0), np.uint32(k1),
          np.uint32(k0) ^ np.uint32(k1) ^ np.uint32(0x1BD11BDA)]
    x0 = (x0 + ks[0]).astype(np.uint32)
    x1 = (x1 + ks[1]).astype(np.uint32)
    rot = [[13, 15, 26, 6], [17, 29, 16, 24]]
    for i in range(5):
        for r in rot[i % 2]:
            x0 = (x0 + x1).astype(np.uint32)
            x1 = _rotl(x1, r)
            x1 = (x1 ^ x0).astype(np.uint32)
        x0 = (x0 + ks[(i + 1) % 3]).astype(np.uint32)
        x1 = (x1 + ks[(i + 2) % 3] + np.uint32(i + 1)).astype(np.uint32)
    return x0, x1


def _race_offsets():
    i64 = np.arange(_ROWS * _VOCAB, dtype=np.uint64)
    b1, b2 = _threefry2x32(0, 1234,
                           (i64 >> np.uint64(32)).astype(np.uint32),
                           (i64 & np.uint64(0xFFFFFFFF)).astype(np.uint32))
    bits = (b1 ^ b2).astype(np.uint32)
    fb = (bits >> np.uint32(9)) | np.uint32(0x3F800000)
    u = np.maximum(np.float32(0.0), fb.view(np.float32) - np.float32(1.0))
    noise = np.maximum(-np.log1p(-u), np.float32(1e-10))
    return (-np.log(noise.astype(np.float64))).astype(np.float32).reshape(
        _ROWS, _VOCAB)


_G = _race_offsets()


def _quantize_g():
    lo = _G.min(axis=1)
    hi = _G.max(axis=1)
    scale = ((hi - lo).astype(np.float64) / 65534.0).astype(np.float32)
    q = np.ceil((_G - lo[:, None]).astype(np.float64)
                / scale[:, None].astype(np.float64)).astype(np.int64)
    q = np.clip(q, 0, 65534)
    for _ in range(2):
        dec = (lo[:, None] + scale[:, None]
               * q.astype(np.float32)).astype(np.float32)
        q = np.where(dec < _G, q + 1, q)
        q = np.clip(q, 0, 65535)
    dec = (lo[:, None] + scale[:, None]
           * q.astype(np.float32)).astype(np.float32)
    emax = (dec.astype(np.float64) - _G.astype(np.float64)).max(axis=1)
    emax = np.nextafter(emax.astype(np.float32), np.float32(np.inf))
    return (q.astype(np.uint16), lo.reshape(-1, 1), scale.reshape(-1, 1),
            emax)


_G16, _GLO, _GSCALE, _EMAX = _quantize_g()


def _scan_body(t_ref, lo_ref, sc_ref, x_ref, q_ref,
               om_ref, oi_ref, os_ref,
               m1_sc, i1_sc, m2_sc):
    j = pl.program_id(0)
    t = t_ref[:, :]
    invt = 1.0 / jnp.where(t == 0.0, 1.0, t)
    gscale = jnp.where(jnp.all(t == 0.0), 0.0, 1.0)
    glo = lo_ref[:, :] * gscale
    gsc = sc_ref[:, :] * gscale

    xi = x_ref[:, :] * invt
    qf = q_ref[:, :].astype(jnp.float32)
    col = jax.lax.broadcasted_iota(jnp.int32, xi.shape, 1)
    u = xi + (qf * gsc + glo)
    u = jnp.where(col + j * _CHUNK < _VOCAB, u, _NEG_INF)

    bm = jnp.max(u, axis=1, keepdims=True)
    ba = jnp.min(jnp.where(u == bm, col, _BIG_I32),
                 axis=1, keepdims=True)
    bs = jnp.max(jnp.where(col == ba, _NEG_INF, u),
                 axis=1, keepdims=True)
    ba = ba + j * _CHUNK

    @pl.when(j == 0)
    def _():
        m1_sc[:, :] = bm
        i1_sc[:, :] = ba
        m2_sc[:, :] = bs

    @pl.when(j > 0)
    def _():
        m1 = m1_sc[:, :]
        upd = bm > m1
        m2_sc[:, :] = jnp.where(upd, jnp.maximum(m1, bs),
                                jnp.maximum(m2_sc[:, :], bm))
        m1_sc[:, :] = jnp.where(upd, bm, m1)
        i1_sc[:, :] = jnp.where(upd, ba, i1_sc[:, :])

    @pl.when(j == _GRID - 1)
    def _():
        om_ref[:, :] = m1_sc[:, :]
        oi_ref[:, :] = i1_sc[:, :]
        os_ref[:, :] = m2_sc[:, :]


def _race_scan(t2, logits):
    return pl.pallas_call(
        _scan_body,
        grid=(_GRID,),
        in_specs=[
            pl.BlockSpec((_ROWS, 1), lambda j: (0, 0)),
            pl.BlockSpec((_ROWS, 1), lambda j: (0, 0)),
            pl.BlockSpec((_ROWS, 1), lambda j: (0, 0)),
            pl.BlockSpec((_ROWS, _CHUNK), lambda j: (0, j)),
            pl.BlockSpec((_ROWS, _CHUNK), lambda j: (0, j)),
        ],
        out_specs=[
            pl.BlockSpec((_ROWS, 1), lambda j: (0, 0)),
            pl.BlockSpec((_ROWS, 1), lambda j: (0, 0)),
            pl.BlockSpec((_ROWS, 1), lambda j: (0, 0)),
        ],
        out_shape=[
            jax.ShapeDtypeStruct((_ROWS, 1), jnp.float32),
            jax.ShapeDtypeStruct((_ROWS, 1), jnp.int32),
            jax.ShapeDtypeStruct((_ROWS, 1), jnp.float32),
        ],
        scratch_shapes=[
            pltpu.VMEM((_ROWS, 1), jnp.float32),
            pltpu.VMEM((_ROWS, 1), jnp.int32),
            pltpu.VMEM((_ROWS, 1), jnp.float32),
        ],
    )(t2, jnp.asarray(_GLO), jnp.asarray(_GSCALE), logits, jnp.asarray(_G16))


def _race_body(t_ref, x_ref, g_ref, o_ref, m_sc, i_sc):
    j = pl.program_id(0)
    t = t_ref[:, :]
    invt = 1.0 / jnp.where(t == 0.0, 1.0, t)
    gscale = jnp.where(jnp.all(t == 0.0), 0.0, 1.0)
    x = x_ref[:, :]
    g = g_ref[:, :]
    col = jax.lax.broadcasted_iota(jnp.int32, x.shape, 1)
    val = x * invt + g * gscale
    val = jnp.where(col + j * _CHUNK < _VOCAB, val, _NEG_INF)
    bmax = jnp.max(val, axis=1, keepdims=True)
    barg = jnp.min(jnp.where(val == bmax, col, _BIG_I32),
                   axis=1, keepdims=True) + j * _CHUNK

    @pl.when(j == 0)
    def _():
        m_sc[:, :] = jnp.full_like(bmax, _NEG_INF)
        i_sc[:, :] = jnp.zeros_like(barg)

    upd = bmax > m_sc[:, :]
    m_sc[:, :] = jnp.where(upd, bmax, m_sc[:, :])
    i_sc[:, :] = jnp.where(upd, barg, i_sc[:, :])

    @pl.when(j == _GRID - 1)
    def _():
        o_ref[:, :] = i_sc[:, :]


def _dense_race(t2, logits):
    out = pl.pallas_call(
        _race_body,
        grid=(_GRID,),
        in_specs=[
            pl.BlockSpec((_ROWS, 1), lambda j: (0, 0)),
            pl.BlockSpec((_ROWS, _CHUNK), lambda j: (0, j)),
            pl.BlockSpec((_ROWS, _CHUNK), lambda j: (0, j)),
        ],
        out_specs=pl.BlockSpec((_ROWS, 1), lambda j: (0, 0)),
        out_shape=jax.ShapeDtypeStruct((_ROWS, 1), jnp.int32),
        scratch_shapes=[
            pltpu.VMEM((_ROWS, 1), jnp.float32),
            pltpu.VMEM((_ROWS, 1), jnp.int32),
        ],
    )(t2, logits, jnp.asarray(_G))
    return out[:, 0]


def kernel(logits, temperatures):
    t = temperatures.astype(jnp.float32)
    t2 = t.reshape(_ROWS, 1)
    u1, i1, u2 = _race_scan(t2, logits)
    idx = i1[:, 0]
    margin = jnp.asarray(_EMAX) + 1e-4 + 1e-5 * jnp.abs(u1[:, 0])
    safe = jnp.all(u1[:, 0] - u2[:, 0] > margin)
    return lax.cond(safe,
                    lambda: idx,
                    lambda: _dense_race(t2, logits))


# speedup vs baseline: 1.2843x; 1.0487x over previous
import jax
import jax.numpy as jnp
import numpy as np
from jax.experimental import pallas as pl
from jax.experimental.pallas import tpu as pltpu

_ROWS, _VOCAB = 64, 100000
_CHUNK = 25600
_GRID = (_VOCAB + _CHUNK - 1) // _CHUNK
_FROWS = 8
_FSTEPS = _ROWS // _FROWS
_NEG_INF = float(np.finfo(np.float32).min)
_BIG_I32 = np.int32(2**31 - 1)


def _rotl(x, r):
    return ((x << np.uint32(r)) | (x >> np.uint32(32 - r))).astype(np.uint32)


def _threefry2x32(k0, k1, x0, x1):
    ks = [np.uint32(k0), np.uint32(k1),
          np.uint32(k0) ^ np.uint32(k1) ^ np.uint32(0x1BD11BDA)]
    x0 = (x0 + ks[0]).astype(np.uint32)
    x1 = (x1 + ks[1]).astype(np.uint32)
    rot = [[13, 15, 26, 6], [17, 29, 16, 24]]
    for i in range(5):
        for r in rot[i % 2]:
            x0 = (x0 + x1).astype(np.uint32)
            x1 = _rotl(x1, r)
            x1 = (x1 ^ x0).astype(np.uint32)
        x0 = (x0 + ks[(i + 1) % 3]).astype(np.uint32)
        x1 = (x1 + ks[(i + 2) % 3] + np.uint32(i + 1)).astype(np.uint32)
    return x0, x1


def _race_offsets():
    i64 = np.arange(_ROWS * _VOCAB, dtype=np.uint64)
    b1, b2 = _threefry2x32(0, 1234,
                           (i64 >> np.uint64(32)).astype(np.uint32),
                           (i64 & np.uint64(0xFFFFFFFF)).astype(np.uint32))
    bits = (b1 ^ b2).astype(np.uint32)
    fb = (bits >> np.uint32(9)) | np.uint32(0x3F800000)
    u = np.maximum(np.float32(0.0), fb.view(np.float32) - np.float32(1.0))
    noise = np.maximum(-np.log1p(-u), np.float32(1e-10))
    return (-np.log(noise.astype(np.float64))).astype(np.float32).reshape(
        _ROWS, _VOCAB)


_G = _race_offsets()


def _quantize_g():
    lo = _G.min(axis=1)
    hi = _G.max(axis=1)
    scale = ((hi - lo).astype(np.float64) / 65534.0).astype(np.float32)
    q = np.ceil((_G - lo[:, None]).astype(np.float64)
                / scale[:, None].astype(np.float64)).astype(np.int64)
    q = np.clip(q, 0, 65534)
    for _ in range(2):
        dec = (lo[:, None] + scale[:, None]
               * q.astype(np.float32)).astype(np.float32)
        q = np.where(dec < _G, q + 1, q)
        q = np.clip(q, 0, 65535)
    dec = (lo[:, None] + scale[:, None]
           * q.astype(np.float32)).astype(np.float32)
    emax = (dec.astype(np.float64) - _G.astype(np.float64)).max(axis=1)
    emax = np.nextafter(emax.astype(np.float32), np.float32(np.inf))
    return (q.astype(np.uint16), lo.reshape(-1, 1), scale.reshape(-1, 1),
            emax.reshape(-1, 1))


_G16, _GLO, _GSCALE, _EMAX = _quantize_g()


def _best_of_block(val, col_off):
    bm = jnp.max(val, axis=1, keepdims=True)
    col = jax.lax.broadcasted_iota(jnp.int32, val.shape, 1)
    ba = jnp.min(jnp.where(val == bm, col, _BIG_I32),
                 axis=1, keepdims=True) + col_off
    return bm, ba, col


def _scan_body(t_ref, lo_ref, sc_ref, em_ref, x_ref, q_ref,
               xf_ref, gf_ref, oi_ref,
               m1_sc, i1_sc, m2_sc, fx_sc, fg_sc, sx_sem, sg_sem):
    j = pl.program_id(0)
    t = t_ref[:, :]
    invt = 1.0 / jnp.where(t == 0.0, 1.0, t)
    gscale = jnp.where(jnp.all(t == 0.0), 0.0, 1.0)
    glo = lo_ref[:, :] * gscale
    gsc = sc_ref[:, :] * gscale

    xi = x_ref[:, :] * invt
    qf = q_ref[:, :].astype(jnp.float32)
    u = xi + (qf * gsc + glo)
    col = jax.lax.broadcasted_iota(jnp.int32, u.shape, 1)
    u = jnp.where(col + j * _CHUNK < _VOCAB, u, _NEG_INF)
    bm, ba, _ = _best_of_block(u, j * _CHUNK)
    bs = jnp.max(jnp.where(col + j * _CHUNK == ba, _NEG_INF, u),
                 axis=1, keepdims=True)

    @pl.when(j == 0)
    def _():
        m1_sc[:, :] = bm
        i1_sc[:, :] = ba
        m2_sc[:, :] = bs

    @pl.when(j > 0)
    def _():
        m1 = m1_sc[:, :]
        upd = bm > m1
        m2_sc[:, :] = jnp.where(upd, jnp.maximum(m1, bs),
                                jnp.maximum(m2_sc[:, :], bm))
        m1_sc[:, :] = jnp.where(upd, bm, m1)
        i1_sc[:, :] = jnp.where(upd, ba, i1_sc[:, :])

    @pl.when(j == _GRID - 1)
    def _():
        u1 = m1_sc[:, :]
        margin = em_ref[:, :] + 1e-4 + 1e-5 * jnp.abs(u1)
        safe = jnp.all(u1 - m2_sc[:, :] > margin)

        @pl.when(safe)
        def _():
            oi_ref[:, :] = i1_sc[:, :]

        @pl.when(jnp.logical_not(safe))
        def _():
            for k in range(_FSTEPS):
                rows = pl.ds(k * _FROWS, _FROWS)
                cx = pltpu.make_async_copy(xf_ref.at[rows, :], fx_sc, sx_sem)
                cg = pltpu.make_async_copy(gf_ref.at[rows, :], fg_sc, sg_sem)
                cx.start()
                cg.start()
                cx.wait()
                cg.wait()
                val = (fx_sc[:, :] * invt[k * _FROWS:(k + 1) * _FROWS, :]
                       + fg_sc[:, :] * gscale)
                fm, fa, _ = _best_of_block(val, 0)
                i1_sc[rows, :] = fa
            oi_ref[:, :] = i1_sc[:, :]


def kernel(logits, temperatures):
    t2 = temperatures.reshape(_ROWS, 1).astype(jnp.float32)
    out = pl.pallas_call(
        _scan_body,
        grid=(_GRID,),
        in_specs=[
            pl.BlockSpec((_ROWS, 1), lambda j: (0, 0)),
            pl.BlockSpec((_ROWS, 1), lambda j: (0, 0)),
            pl.BlockSpec((_ROWS, 1), lambda j: (0, 0)),
            pl.BlockSpec((_ROWS, 1), lambda j: (0, 0)),
            pl.BlockSpec((_ROWS, _CHUNK), lambda j: (0, j)),
            pl.BlockSpec((_ROWS, _CHUNK), lambda j: (0, j)),
            pl.BlockSpec(memory_space=pl.ANY),
            pl.BlockSpec(memory_space=pl.ANY),
        ],
        out_specs=pl.BlockSpec((_ROWS, 1), lambda j: (0, 0)),
        out_shape=jax.ShapeDtypeStruct((_ROWS, 1), jnp.int32),
        scratch_shapes=[
            pltpu.VMEM((_ROWS, 1), jnp.float32),
            pltpu.VMEM((_ROWS, 1), jnp.int32),
            pltpu.VMEM((_ROWS, 1), jnp.float32),
            pltpu.VMEM((_FROWS, _VOCAB), jnp.float32),
            pltpu.VMEM((_FROWS, _VOCAB), jnp.float32),
            pltpu.SemaphoreType.DMA,
            pltpu.SemaphoreType.DMA,
        ],
    )(t2, jnp.asarray(_GLO), jnp.asarray(_GSCALE), jnp.asarray(_EMAX),
      logits, jnp.asarray(_G16), logits, jnp.asarray(_G))
    return out[:, 0]
